# baseline (speedup 1.0000x reference)
import functools

import jax
import jax.numpy as jnp
from jax import lax
from jax.experimental import pallas as pl
from jax.experimental.pallas import tpu as pltpu
from jax.experimental.pallas import tpu_sc as plsc

SEQ_NUM = 1000000
N_EMBD = 64
WIN_LEN = 200
BATCH = 4096
INPUT_DIM = 64
OUT_DIM = INPUT_DIM + N_EMBD

_info = plsc.get_sparse_core_info()
_NC, _NS, _L = _info.num_cores, _info.num_subcores, _info.num_lanes
_NW = _NC * _NS
_B_PER_W = BATCH // _NW


def _sc_gather(idx, table2):
    mesh = plsc.VectorSubcoreMesh(core_axis_name="c", subcore_axis_name="s")

    @functools.partial(
        pl.kernel,
        mesh=mesh,
        out_type=jax.ShapeDtypeStruct((BATCH, 2 * N_EMBD), jnp.float32),
        scratch_types=[
            pltpu.VMEM((_B_PER_W,), jnp.int32),
            pltpu.VMEM((_B_PER_W,), jnp.int32),
            pltpu.VMEM((_B_PER_W, 2 * N_EMBD), jnp.float32),
            pltpu.SemaphoreType.DMA,
        ],
    )
    def k(idx_hbm, table_hbm, out_hbm, idx_v, half_v, rows_v, sem):
        wid = lax.axis_index("s") * _NC + lax.axis_index("c")
        base = wid * _B_PER_W
        pltpu.sync_copy(idx_hbm.at[pl.ds(base, _B_PER_W)], idx_v)
        for kk in range(_B_PER_W // _L):
            sl = pl.ds(kk * _L, _L)
            half_v[sl] = lax.shift_right_logical(idx_v[sl], 1)
        pltpu.async_copy(table_hbm.at[half_v], rows_v, sem).wait()
        pltpu.sync_copy(rows_v, out_hbm.at[pl.ds(base, _B_PER_W)])

    return k(idx, table2)


_BB = 64
_NB = BATCH // _BB


def _tc_body(x_ref, g_ref, sid_ref, po_ref, o_ref):
    par = (sid_ref[0, :, :] & 1).astype(jnp.bool_)
    emb = jnp.where(par, g_ref[:, N_EMBD:], g_ref[:, :N_EMBD])
    o_ref[:, :, 0:INPUT_DIM] = x_ref[...]
    o_ref[:, :, INPUT_DIM:] = emb[:, None, :] + po_ref[...][None, :, :]


def _tc_concat(x, gathered, sid3, po_table):
    return pl.pallas_call(
        _tc_body,
        grid=(_NB,),
        in_specs=[
            pl.BlockSpec((_BB, WIN_LEN, INPUT_DIM), lambda i: (i, 0, 0)),
            pl.BlockSpec((_BB, 2 * N_EMBD), lambda i: (i, 0)),
            pl.BlockSpec((1, _BB, 1), lambda i: (i, 0, 0)),
            pl.BlockSpec((WIN_LEN, N_EMBD), lambda i: (0, 0)),
        ],
        out_specs=pl.BlockSpec((_BB, WIN_LEN, OUT_DIM), lambda i: (i, 0, 0)),
        out_shape=jax.ShapeDtypeStruct((BATCH, WIN_LEN, OUT_DIM), jnp.float32),
    )(x, gathered, sid3, po_table)


@jax.jit
def kernel(series_id, x, id_table, po_table):
    sid = series_id.astype(jnp.int32)
    table2 = id_table.reshape(SEQ_NUM // 2, 2 * N_EMBD)
    gathered = _sc_gather(sid, table2)
    sid3 = sid.reshape(_NB, _BB, 1)
    return _tc_concat(x, gathered, sid3, po_table)


# speedup vs baseline: 3.7738x; 3.7738x over previous
import functools

import jax
import jax.numpy as jnp
from jax import lax
from jax.experimental import pallas as pl
from jax.experimental.pallas import tpu as pltpu
from jax.experimental.pallas import tpu_sc as plsc

SEQ_NUM = 1000000
N_EMBD = 64
WIN_LEN = 200
BATCH = 4096
INPUT_DIM = 64
OUT_DIM = INPUT_DIM + N_EMBD

_info = plsc.get_sparse_core_info()
_NC, _NS, _L = _info.num_cores, _info.num_subcores, _info.num_lanes
_NW = _NC * _NS
_B_PER_W = BATCH // _NW
_WIN = 8


def _sc_gather(idx, tableT):
    mesh = plsc.VectorSubcoreMesh(core_axis_name="c", subcore_axis_name="s")

    @functools.partial(
        pl.kernel,
        mesh=mesh,
        out_type=jax.ShapeDtypeStruct((BATCH, N_EMBD), jnp.float32),
        scratch_types=[
            pltpu.VMEM((_B_PER_W,), jnp.int32),
            pltpu.VMEM((_B_PER_W, N_EMBD), jnp.float32),
            pltpu.VMEM((_WIN, N_EMBD, 128), jnp.float32),
            pltpu.SemaphoreType.DMA,
        ],
        compiler_params=pltpu.CompilerParams(needs_layout_passes=False),
    )
    def k(idx_hbm, t_hbm, out_hbm, idx_v, rows_v, tiles_v, sem):
        wid = lax.axis_index("s") * _NC + lax.axis_index("c")
        base = wid * _B_PER_W
        pltpu.sync_copy(idx_hbm.at[pl.ds(base, _B_PER_W)], idx_v)

        n_grp = _B_PER_W // _L
        ci = lax.iota(jnp.int32, _L)

        def issue(slot, r):
            rt = pl.multiple_of((r >> 7) << 7, 128)
            pltpu.async_copy(t_hbm.at[:, pl.ds(rt, 128)], tiles_v.at[slot], sem)

        def drain(slot):
            pltpu.make_async_copy(
                t_hbm.at[:, pl.ds(0, 128)], tiles_v.at[slot], sem
            ).wait()

        def extract(slot, r, j):
            rl = jnp.broadcast_to(r & 127, (_L,))
            for cg in range(N_EMBD // _L):
                res = plsc.load_gather(tiles_v.at[slot], [cg * _L + ci, rl])
                rows_v[j, pl.ds(cg * _L, _L)] = res

        def group(gi, carry):
            vec = idx_v[pl.ds(gi * _L, _L)]
            for kk in range(_WIN):
                issue(kk, vec[kk])
            for kk in range(_WIN, _L):
                s = kk - _WIN
                drain(s)
                extract(s, vec[s], gi * _L + s)
                issue(s, vec[kk])
            for kk in range(_L - _WIN, _L):
                s = kk - (_L - _WIN)
                drain(s)
                extract(s, vec[kk], gi * _L + kk)
            return carry

        lax.fori_loop(0, n_grp, group, 0)
        pltpu.sync_copy(rows_v, out_hbm.at[pl.ds(base, _B_PER_W)])

    return k(idx, tableT)


_BBT = 128
_NBT = BATCH // _BBT


def _tc_body(x_ref, g_ref, po_ref, o_ref):
    xv = x_ref[...]
    xt = jnp.transpose(xv, (2, 0, 1))
    o_ref[:, :, 0:INPUT_DIM] = xt
    o_ref[:, :, INPUT_DIM:] = g_ref[...][:, None, :] + po_ref[...][None, :, :]


def _tc_concat(xT, g, po_table):
    return pl.pallas_call(
        _tc_body,
        grid=(_NBT,),
        in_specs=[
            pl.BlockSpec((WIN_LEN, INPUT_DIM, _BBT), lambda i: (0, 0, i)),
            pl.BlockSpec((_BBT, N_EMBD), lambda i: (i, 0)),
            pl.BlockSpec((WIN_LEN, N_EMBD), lambda i: (0, 0)),
        ],
        out_specs=pl.BlockSpec((_BBT, WIN_LEN, OUT_DIM), lambda i: (i, 0, 0)),
        out_shape=jax.ShapeDtypeStruct((BATCH, WIN_LEN, OUT_DIM), jnp.float32),
    )(xT, g, po_table)


@jax.jit
def kernel(series_id, x, id_table, po_table):
    sid = series_id.astype(jnp.int32)
    tableT = id_table.T
    xT = jnp.transpose(x, (1, 2, 0))
    g = _sc_gather(sid, tableT)
    return _tc_concat(xT, g, po_table)


# speedup vs baseline: 3.8630x; 1.0236x over previous
import functools

import jax
import jax.numpy as jnp
from jax import lax
from jax.experimental import pallas as pl
from jax.experimental.pallas import tpu as pltpu
from jax.experimental.pallas import tpu_sc as plsc

SEQ_NUM = 1000000
N_EMBD = 64
WIN_LEN = 200
BATCH = 4096
INPUT_DIM = 64
OUT_DIM = INPUT_DIM + N_EMBD

_info = plsc.get_sparse_core_info()
_NC, _NS, _L = _info.num_cores, _info.num_subcores, _info.num_lanes
_NW = _NC * _NS
_B_PER_W = BATCH // _NW
_WIN = 8


def _sc_gather(idx, tableT):
    n = idx.shape[0]
    b_per_w = n // _NW
    mesh = plsc.VectorSubcoreMesh(core_axis_name="c", subcore_axis_name="s")

    @functools.partial(
        pl.kernel,
        mesh=mesh,
        out_type=jax.ShapeDtypeStruct((n, N_EMBD), jnp.float32),
        scratch_types=[
            pltpu.VMEM((b_per_w,), jnp.int32),
            pltpu.VMEM((b_per_w, N_EMBD), jnp.float32),
            pltpu.VMEM((_WIN, N_EMBD, 128), jnp.float32),
            pltpu.SemaphoreType.DMA,
        ],
        compiler_params=pltpu.CompilerParams(needs_layout_passes=False),
    )
    def k(idx_hbm, t_hbm, out_hbm, idx_v, rows_v, tiles_v, sem):
        wid = lax.axis_index("s") * _NC + lax.axis_index("c")
        base = wid * b_per_w
        pltpu.sync_copy(idx_hbm.at[pl.ds(base, b_per_w)], idx_v)

        n_grp = b_per_w // _L
        ci = lax.iota(jnp.int32, _L)

        def issue(slot, r):
            rt = pl.multiple_of((r >> 7) << 7, 128)
            pltpu.async_copy(t_hbm.at[:, pl.ds(rt, 128)], tiles_v.at[slot], sem)

        def drain(slot):
            pltpu.make_async_copy(
                t_hbm.at[:, pl.ds(0, 128)], tiles_v.at[slot], sem
            ).wait()

        def extract(slot, r, j):
            rl = jnp.broadcast_to(r & 127, (_L,))
            for cg in range(N_EMBD // _L):
                res = plsc.load_gather(tiles_v.at[slot], [cg * _L + ci, rl])
                rows_v[j, pl.ds(cg * _L, _L)] = res

        def group(gi, carry):
            vec = idx_v[pl.ds(gi * _L, _L)]
            for kk in range(_WIN):
                issue(kk, vec[kk])
            for kk in range(_WIN, _L):
                s = kk - _WIN
                drain(s)
                extract(s, vec[s], gi * _L + s)
                issue(s, vec[kk])
            for kk in range(_L - _WIN, _L):
                s = kk - (_L - _WIN)
                drain(s)
                extract(s, vec[kk], gi * _L + kk)
            return carry

        lax.fori_loop(0, n_grp, group, 0)
        pltpu.sync_copy(rows_v, out_hbm.at[pl.ds(base, b_per_w)])

    return k(idx, tableT)


_BBT = 128
_NBT = BATCH // _BBT
_OUT_SHAPE = jax.ShapeDtypeStruct((BATCH, WIN_LEN, OUT_DIM), jnp.float32)


_HALF = _NBT // 2


def _tc_body(x_ref, g_ref, po_ref, o_ref):
    xv = x_ref[...]
    o_ref[:, :, 0:INPUT_DIM] = jnp.transpose(xv, (2, 0, 1))
    o_ref[:, :, INPUT_DIM:] = g_ref[...][:, None, :] + po_ref[...][None, :, :]


def _tc_concat_half(xT, g_half, po_table, half, prev=None):
    base = half * _HALF

    def _body(*refs):
        if prev is None:
            _tc_body(*refs)
        else:
            _tc_body(*refs[1:])

    in_specs = [
        pl.BlockSpec((WIN_LEN, INPUT_DIM, _BBT), lambda i: (0, 0, base + i)),
        pl.BlockSpec((_BBT, N_EMBD), lambda i: (i, 0)),
        pl.BlockSpec((WIN_LEN, N_EMBD), lambda i: (0, 0)),
    ]
    args = (xT, g_half, po_table)
    kwargs = {}
    if prev is not None:
        in_specs = [pl.BlockSpec(memory_space=pl.ANY)] + in_specs
        args = (prev,) + args
        kwargs = dict(input_output_aliases={0: 0})
    return pl.pallas_call(
        _body,
        grid=(_HALF,),
        in_specs=in_specs,
        out_specs=pl.BlockSpec((_BBT, WIN_LEN, OUT_DIM), lambda i: (base + i, 0, 0)),
        out_shape=_OUT_SHAPE,
        **kwargs,
    )(*args)


@jax.jit
def kernel(series_id, x, id_table, po_table):
    sid = series_id.astype(jnp.int32)
    tableT = id_table.T
    xT = jnp.transpose(x, (1, 2, 0))
    g0 = _sc_gather(sid[: BATCH // 2], tableT)
    g1 = _sc_gather(sid[BATCH // 2 :], tableT)
    out0 = _tc_concat_half(xT, g0, po_table, 0)
    return _tc_concat_half(xT, g1, po_table, 1, prev=out0)


# speedup vs baseline: 3.8960x; 1.0085x over previous
import functools

import jax
import jax.numpy as jnp
from jax import lax
from jax.experimental import pallas as pl
from jax.experimental.pallas import tpu as pltpu
from jax.experimental.pallas import tpu_sc as plsc

SEQ_NUM = 1000000
N_EMBD = 64
WIN_LEN = 200
BATCH = 4096
INPUT_DIM = 64
OUT_DIM = INPUT_DIM + N_EMBD

_info = plsc.get_sparse_core_info()
_NC, _NS, _L = _info.num_cores, _info.num_subcores, _info.num_lanes
_NW = _NC * _NS
_B_PER_W = BATCH // _NW
_WIN = 8


def _sc_gather(idx, tableT):
    n = idx.shape[0]
    b_per_w = n // _NW
    mesh = plsc.VectorSubcoreMesh(core_axis_name="c", subcore_axis_name="s")

    @functools.partial(
        pl.kernel,
        mesh=mesh,
        out_type=jax.ShapeDtypeStruct((n, N_EMBD), jnp.float32),
        scratch_types=[
            pltpu.VMEM((b_per_w,), jnp.int32),
            pltpu.VMEM((b_per_w, N_EMBD), jnp.float32),
            pltpu.VMEM((_WIN, N_EMBD, 128), jnp.float32),
            pltpu.SemaphoreType.DMA,
        ],
        compiler_params=pltpu.CompilerParams(needs_layout_passes=False),
    )
    def k(idx_hbm, t_hbm, out_hbm, idx_v, rows_v, tiles_v, sem):
        wid = lax.axis_index("s") * _NC + lax.axis_index("c")
        base = wid * b_per_w
        pltpu.sync_copy(idx_hbm.at[pl.ds(base, b_per_w)], idx_v)

        n_grp = b_per_w // _L
        ci = lax.iota(jnp.int32, _L)

        def issue(slot, r):
            rt = pl.multiple_of((r >> 7) << 7, 128)
            pltpu.async_copy(t_hbm.at[:, pl.ds(rt, 128)], tiles_v.at[slot], sem)

        def drain(slot):
            pltpu.make_async_copy(
                t_hbm.at[:, pl.ds(0, 128)], tiles_v.at[slot], sem
            ).wait()

        def extract(slot, r, j):
            rl = jnp.broadcast_to(r & 127, (_L,))
            for cg in range(N_EMBD // _L):
                res = plsc.load_gather(tiles_v.at[slot], [cg * _L + ci, rl])
                rows_v[j, pl.ds(cg * _L, _L)] = res

        def group(gi, carry):
            vec = idx_v[pl.ds(gi * _L, _L)]
            for kk in range(_WIN):
                issue(kk, vec[kk])
            for kk in range(_WIN, _L):
                s = kk - _WIN
                drain(s)
                extract(s, vec[s], gi * _L + s)
                issue(s, vec[kk])
            for kk in range(_L - _WIN, _L):
                s = kk - (_L - _WIN)
                drain(s)
                extract(s, vec[kk], gi * _L + kk)
            return carry

        lax.fori_loop(0, n_grp, group, 0)
        pltpu.sync_copy(rows_v, out_hbm.at[pl.ds(base, b_per_w)])

    return k(idx, tableT)


_BBT = 128
_NBT = BATCH // _BBT
_OUT_SHAPE = jax.ShapeDtypeStruct((BATCH, WIN_LEN, OUT_DIM), jnp.float32)


_SPLIT = 1024


def _tc_body(x_ref, g_ref, po_ref, o_ref):
    xv = x_ref[...]
    o_ref[:, :, 0:INPUT_DIM] = jnp.transpose(xv, (2, 0, 1))
    o_ref[:, :, INPUT_DIM:] = g_ref[...][:, None, :] + po_ref[...][None, :, :]


def _tc_concat_part(xT, g_part, po_table, row0, prev=None):
    base = row0 // _BBT
    steps = g_part.shape[0] // _BBT

    def _body(*refs):
        if prev is None:
            _tc_body(*refs)
        else:
            _tc_body(*refs[1:])

    in_specs = [
        pl.BlockSpec((WIN_LEN, INPUT_DIM, _BBT), lambda i: (0, 0, base + i)),
        pl.BlockSpec((_BBT, N_EMBD), lambda i: (i, 0)),
        pl.BlockSpec((WIN_LEN, N_EMBD), lambda i: (0, 0)),
    ]
    args = (xT, g_part, po_table)
    kwargs = {}
    if prev is not None:
        in_specs = [pl.BlockSpec(memory_space=pl.ANY)] + in_specs
        args = (prev,) + args
        kwargs = dict(input_output_aliases={0: 0})
    return pl.pallas_call(
        _body,
        grid=(steps,),
        in_specs=in_specs,
        out_specs=pl.BlockSpec((_BBT, WIN_LEN, OUT_DIM), lambda i: (base + i, 0, 0)),
        out_shape=_OUT_SHAPE,
        **kwargs,
    )(*args)


@jax.jit
def kernel(series_id, x, id_table, po_table):
    sid = series_id.astype(jnp.int32)
    tableT = id_table.T
    xT = jnp.transpose(x, (1, 2, 0))
    g0 = _sc_gather(sid[:_SPLIT], tableT)
    g1 = _sc_gather(sid[_SPLIT:], tableT)
    out0 = _tc_concat_part(xT, g0, po_table, 0)
    return _tc_concat_part(xT, g1, po_table, _SPLIT, prev=out0)
